# baseline (device time: 132912 ns/iter reference)
import jax
import jax.numpy as jnp
from jax import lax
from jax.experimental import pallas as pl
from jax.experimental.pallas import tpu as pltpu

N_DEV = 4


def kernel(x, w_mat):
    k_tot, k_blk = x.shape
    _, n_tot = w_mat.shape
    m_blk = k_tot // N_DEV
    n_tile = 1024
    n_tiles = n_tot // n_tile

    def body(x_hbm, w_hbm, out_ref,
             xstage_ref, send_ref, comm_ref, y_ref, wbuf_ref,
             amax_src_ref, amax_comm_ref,
             send_sems, recv_sems, amax_send_sems, amax_recv_sems,
             xl_sem, w_sems):
        me = lax.axis_index("i")

        scope_entry = jax.named_scope("entry")
        scope_entry.__enter__()
        barrier = pltpu.get_barrier_semaphore()
        for t in range(1, N_DEV):
            peer = lax.rem(me + t, N_DEV)
            pl.semaphore_signal(barrier, inc=1, device_id=(peer,),
                                device_id_type=pl.DeviceIdType.MESH)
        pl.semaphore_wait(barrier, N_DEV - 1)

        def xload(t):
            rows = lax.rem(me + t, N_DEV) * m_blk
            return pltpu.make_async_copy(
                x_hbm.at[pl.ds(rows, m_blk), :], xstage_ref, xl_sem)

        def make_rdma(t):
            target = lax.rem(me + t, N_DEV)
            slot = N_DEV - t
            return pltpu.make_async_remote_copy(
                src_ref=send_ref.at[t],
                dst_ref=comm_ref.at[slot - 1],
                send_sem=send_sems.at[t],
                recv_sem=recv_sems.at[slot],
                device_id=(target,),
                device_id_type=pl.DeviceIdType.MESH,
            )

        xload(1).start()
        xload(1).wait()
        send_ref[1] = xstage_ref[...].astype(jnp.bfloat16)
        rdma1 = make_rdma(1)
        rdma1.start()

        xload(3).start()
        xload(3).wait()
        send_ref[3] = xstage_ref[...].astype(jnp.bfloat16)
        rdma3 = make_rdma(3)
        rdma3.start()

        xload(2).start()
        xload(2).wait()
        send_ref[2] = xstage_ref[...].astype(jnp.bfloat16)

        xload(0).start()
        xload(0).wait()
        send_ref[0] = xstage_ref[...].astype(jnp.bfloat16)
        scope_entry.__exit__(None, None, None)

        def make_wcopy(k_idx, n, buf):
            return pltpu.make_async_copy(
                w_hbm.at[pl.ds(k_idx * k_blk, k_blk),
                         pl.ds(n * n_tile, n_tile)],
                wbuf_ref.at[buf], w_sems.at[buf])

        def sweep(slot, first, last, amax_in):
            k_idx = lax.rem(me + slot, N_DEV)
            xsrc = send_ref.at[0] if slot == 0 else comm_ref.at[slot - 1]
            make_wcopy(k_idx, 0, 0).start()

            def step(n, amax):
                buf = lax.rem(n, 2)
                make_wcopy(k_idx, n, buf).wait()

                @pl.when(n + 1 < n_tiles)
                def _():
                    make_wcopy(k_idx, n + 1, 1 - buf).start()

                acc = lax.dot_general(
                    xsrc[...], wbuf_ref[buf].astype(jnp.bfloat16),
                    (((1,), (0,)), ((), ())),
                    preferred_element_type=jnp.float32)
                nds = pl.ds(n * n_tile, n_tile)
                if not first:
                    acc = acc + y_ref[:, nds].astype(jnp.float32)
                if last:
                    acc = jnp.maximum(acc, 0.0)
                    amax = jnp.maximum(amax, jnp.max(acc))
                y_ref[:, nds] = acc.astype(jnp.bfloat16)
                return amax

            return lax.fori_loop(0, n_tiles, step, amax_in)

        with jax.named_scope("sweep0"):
            sweep(0, True, False, jnp.float32(0.0))

        with jax.named_scope("diag_send"):
            rdma2 = make_rdma(2)
            rdma2.start()

        with jax.named_scope("wait_slot3"):
            rdma1.wait_recv()
        with jax.named_scope("sweep3"):
            sweep(3, False, False, jnp.float32(0.0))
        with jax.named_scope("wait_slot1"):
            rdma3.wait_recv()
        with jax.named_scope("sweep1"):
            sweep(1, False, False, jnp.float32(0.0))
        with jax.named_scope("wait_slot2"):
            rdma2.wait_recv()
        with jax.named_scope("sweep2"):
            amax = sweep(2, False, True, jnp.float32(0.0))

        with jax.named_scope("amax_xchg"):
            rdma1.wait_send()
            rdma3.wait_send()
            rdma2.wait_send()

            amax_src_ref[...] = jnp.full((8, 128), amax, jnp.float32)
            amax_comm_ref[0] = amax_src_ref[...]
            a_rdmas = []
            for t in range(1, N_DEV):
                target = lax.rem(me + t, N_DEV)
                slot = N_DEV - t
                r = pltpu.make_async_remote_copy(
                    src_ref=amax_src_ref,
                    dst_ref=amax_comm_ref.at[slot],
                    send_sem=amax_send_sems.at[slot],
                    recv_sem=amax_recv_sems.at[slot],
                    device_id=(target,),
                    device_id_type=pl.DeviceIdType.MESH,
                )
                r.start()
                a_rdmas.append(r)
            for r in a_rdmas:
                r.wait()

        g_amax = jnp.max(amax_comm_ref[...])
        inv = 127.0 / g_amax
        scale = g_amax / 127.0

        with jax.named_scope("quant"):
            def q_step(n, carry):
                nds = pl.ds(n * n_tile, n_tile)
                yt = y_ref[:, nds].astype(jnp.float32)
                q = jnp.clip(jnp.round(yt * inv), -127.0, 127.0)
                out_ref[:, nds] = (q * scale).astype(jnp.bfloat16)
                return carry

            lax.fori_loop(0, n_tiles, q_step, jnp.int32(0))

    return pl.pallas_call(
        body,
        out_shape=jax.ShapeDtypeStruct((m_blk, n_tot), jnp.bfloat16),
        in_specs=[pl.BlockSpec(memory_space=pl.ANY),
                  pl.BlockSpec(memory_space=pl.ANY)],
        out_specs=pl.BlockSpec(memory_space=pltpu.VMEM),
        scratch_shapes=[
            pltpu.VMEM((m_blk, k_blk), jnp.float32),
            pltpu.VMEM((N_DEV, m_blk, k_blk), jnp.bfloat16),
            pltpu.VMEM((3, m_blk, k_blk), jnp.bfloat16),
            pltpu.VMEM((m_blk, n_tot), jnp.bfloat16),
            pltpu.VMEM((2, k_blk, n_tile), jnp.float32),
            pltpu.VMEM((8, 128), jnp.float32),
            pltpu.VMEM((N_DEV, 8, 128), jnp.float32),
            pltpu.SemaphoreType.DMA((N_DEV,)),
            pltpu.SemaphoreType.DMA((N_DEV,)),
            pltpu.SemaphoreType.DMA((N_DEV,)),
            pltpu.SemaphoreType.DMA((N_DEV,)),
            pltpu.SemaphoreType.DMA,
            pltpu.SemaphoreType.DMA((2,)),
        ],
        compiler_params=pltpu.CompilerParams(
            collective_id=0, vmem_limit_bytes=64 * 1024 * 1024),
    )(x, w_mat)


# device time: 131862 ns/iter; 1.0080x vs baseline; 1.0080x over previous
import jax
import jax.numpy as jnp
from jax import lax
from jax.experimental import pallas as pl
from jax.experimental.pallas import tpu as pltpu

N_DEV = 4


def kernel(x, w_mat):
    k_tot, k_blk = x.shape
    _, n_tot = w_mat.shape
    m_blk = k_tot // N_DEV
    n_tile = 1024
    n_tiles = n_tot // n_tile

    def body(x_ref, w_hbm, out_hbm,
             send_ref, comm_ref, y_ref, wbuf_ref, ostage_ref,
             amax_src_ref, amax_comm_ref,
             send_sems, recv_sems, amax_send_sems, amax_recv_sems,
             w_sems, out_sems):
        me = lax.axis_index("i")

        scope_entry = jax.named_scope("entry")
        scope_entry.__enter__()
        barrier = pltpu.get_barrier_semaphore()
        for t in range(1, N_DEV):
            peer = lax.rem(me + t, N_DEV)
            pl.semaphore_signal(barrier, inc=1, device_id=(peer,),
                                device_id_type=pl.DeviceIdType.MESH)
        pl.semaphore_wait(barrier, N_DEV - 1)

        def cvt(t):
            rows = lax.rem(me + t, N_DEV) * m_blk
            send_ref[t] = x_ref[pl.ds(rows, m_blk), :].astype(jnp.bfloat16)

        def make_rdma(t):
            target = lax.rem(me + t, N_DEV)
            slot = N_DEV - t
            return pltpu.make_async_remote_copy(
                src_ref=send_ref.at[t],
                dst_ref=comm_ref.at[slot - 1],
                send_sem=send_sems.at[t],
                recv_sem=recv_sems.at[slot],
                device_id=(target,),
                device_id_type=pl.DeviceIdType.MESH,
            )

        cvt(1)
        rdma1 = make_rdma(1)
        rdma1.start()
        cvt(3)
        rdma3 = make_rdma(3)
        rdma3.start()
        cvt(2)
        cvt(0)
        scope_entry.__exit__(None, None, None)

        def make_wcopy(k_idx, n, buf):
            return pltpu.make_async_copy(
                w_hbm.at[pl.ds(k_idx * k_blk, k_blk),
                         pl.ds(n * n_tile, n_tile)],
                wbuf_ref.at[buf], w_sems.at[buf])

        def sweep(slot, first, last, amax_in):
            k_idx = lax.rem(me + slot, N_DEV)
            xsrc = send_ref.at[0] if slot == 0 else comm_ref.at[slot - 1]
            make_wcopy(k_idx, 0, 0).start()

            def step(n, amax):
                buf = lax.rem(n, 2)
                make_wcopy(k_idx, n, buf).wait()

                @pl.when(n + 1 < n_tiles)
                def _():
                    make_wcopy(k_idx, n + 1, 1 - buf).start()

                acc = lax.dot_general(
                    xsrc[...], wbuf_ref[buf].astype(jnp.bfloat16),
                    (((1,), (0,)), ((), ())),
                    preferred_element_type=jnp.float32)
                nds = pl.ds(n * n_tile, n_tile)
                if not first:
                    acc = acc + y_ref[:, nds].astype(jnp.float32)
                if last:
                    acc = jnp.maximum(acc, 0.0)
                    amax = jnp.maximum(amax, jnp.max(acc))
                y_ref[:, nds] = acc.astype(jnp.bfloat16)
                return amax

            return lax.fori_loop(0, n_tiles, step, amax_in)

        with jax.named_scope("sweep0"):
            sweep(0, True, False, jnp.float32(0.0))

        with jax.named_scope("diag_send"):
            rdma2 = make_rdma(2)
            rdma2.start()

        with jax.named_scope("wait_slot3"):
            rdma1.wait_recv()
        with jax.named_scope("sweep3"):
            sweep(3, False, False, jnp.float32(0.0))
        with jax.named_scope("wait_slot1"):
            rdma3.wait_recv()
        with jax.named_scope("sweep1"):
            sweep(1, False, False, jnp.float32(0.0))
        with jax.named_scope("wait_slot2"):
            rdma2.wait_recv()
        with jax.named_scope("sweep2"):
            amax = sweep(2, False, True, jnp.float32(0.0))

        with jax.named_scope("amax_xchg"):
            rdma1.wait_send()
            rdma3.wait_send()
            rdma2.wait_send()

            amax_src_ref[...] = jnp.full((8, 128), amax, jnp.float32)
            amax_comm_ref[0] = amax_src_ref[...]
            a_rdmas = []
            for t in range(1, N_DEV):
                target = lax.rem(me + t, N_DEV)
                slot = N_DEV - t
                r = pltpu.make_async_remote_copy(
                    src_ref=amax_src_ref,
                    dst_ref=amax_comm_ref.at[slot],
                    send_sem=amax_send_sems.at[slot],
                    recv_sem=amax_recv_sems.at[slot],
                    device_id=(target,),
                    device_id_type=pl.DeviceIdType.MESH,
                )
                r.start()
                a_rdmas.append(r)
            for r in a_rdmas:
                r.wait()

        g_amax = jnp.max(amax_comm_ref[...])
        inv = 127.0 / g_amax
        scale = g_amax / 127.0

        q_tile = 512
        q_tiles = n_tot // q_tile

        def make_ocopy(n, buf):
            return pltpu.make_async_copy(
                ostage_ref.at[buf], out_hbm.at[:, pl.ds(n * q_tile, q_tile)],
                out_sems.at[buf])

        with jax.named_scope("quant"):
            for n in range(q_tiles):
                buf = n % 2
                if n >= 2:
                    make_ocopy(n - 2, buf).wait()
                yt = y_ref[:, pl.ds(n * q_tile, q_tile)].astype(jnp.float32)
                q = jnp.clip(jnp.round(yt * inv), -127.0, 127.0)
                ostage_ref[buf] = (q * scale).astype(jnp.bfloat16)
                make_ocopy(n, buf).start()

            make_ocopy(q_tiles - 2, (q_tiles - 2) % 2).wait()
            make_ocopy(q_tiles - 1, (q_tiles - 1) % 2).wait()

    return pl.pallas_call(
        body,
        out_shape=jax.ShapeDtypeStruct((m_blk, n_tot), jnp.bfloat16),
        in_specs=[pl.BlockSpec(memory_space=pltpu.VMEM),
                  pl.BlockSpec(memory_space=pl.ANY)],
        out_specs=pl.BlockSpec(memory_space=pl.ANY),
        scratch_shapes=[
            pltpu.VMEM((N_DEV, m_blk, k_blk), jnp.bfloat16),
            pltpu.VMEM((3, m_blk, k_blk), jnp.bfloat16),
            pltpu.VMEM((m_blk, n_tot), jnp.bfloat16),
            pltpu.VMEM((2, k_blk, n_tile), jnp.float32),
            pltpu.VMEM((2, m_blk, 512), jnp.bfloat16),
            pltpu.VMEM((8, 128), jnp.float32),
            pltpu.VMEM((N_DEV, 8, 128), jnp.float32),
            pltpu.SemaphoreType.DMA((N_DEV,)),
            pltpu.SemaphoreType.DMA((N_DEV,)),
            pltpu.SemaphoreType.DMA((N_DEV,)),
            pltpu.SemaphoreType.DMA((N_DEV,)),
            pltpu.SemaphoreType.DMA((2,)),
            pltpu.SemaphoreType.DMA((2,)),
        ],
        compiler_params=pltpu.CompilerParams(
            collective_id=0, vmem_limit_bytes=64 * 1024 * 1024),
    )(x, w_mat)


# device time: 131811 ns/iter; 1.0084x vs baseline; 1.0004x over previous
import jax
import jax.numpy as jnp
from jax import lax
from jax.experimental import pallas as pl
from jax.experimental.pallas import tpu as pltpu

N_DEV = 4


def kernel(x, w_mat):
    k_tot, k_blk = x.shape
    _, n_tot = w_mat.shape
    m_blk = k_tot // N_DEV
    n_tile = 1024
    n_tiles = n_tot // n_tile

    def body(x_ref, w_hbm, out_ref,
             send_ref, comm_ref, wbuf_ref,
             amax_src_ref, amax_comm_ref,
             send_sems, recv_sems, amax_send_sems, amax_recv_sems,
             w_sems):
        y_ref = out_ref
        me = lax.axis_index("i")

        scope_entry = jax.named_scope("entry")
        scope_entry.__enter__()
        barrier = pltpu.get_barrier_semaphore()
        for t in range(1, N_DEV):
            peer = lax.rem(me + t, N_DEV)
            pl.semaphore_signal(barrier, inc=1, device_id=(peer,),
                                device_id_type=pl.DeviceIdType.MESH)
        pl.semaphore_wait(barrier, N_DEV - 1)

        def cvt(t):
            rows = lax.rem(me + t, N_DEV) * m_blk
            send_ref[t] = x_ref[pl.ds(rows, m_blk), :].astype(jnp.bfloat16)

        def make_rdma(t):
            target = lax.rem(me + t, N_DEV)
            slot = N_DEV - t
            return pltpu.make_async_remote_copy(
                src_ref=send_ref.at[t],
                dst_ref=comm_ref.at[slot - 1],
                send_sem=send_sems.at[t],
                recv_sem=recv_sems.at[slot],
                device_id=(target,),
                device_id_type=pl.DeviceIdType.MESH,
            )

        cvt(1)
        rdma1 = make_rdma(1)
        rdma1.start()
        cvt(3)
        rdma3 = make_rdma(3)
        rdma3.start()
        cvt(2)
        cvt(0)
        scope_entry.__exit__(None, None, None)

        def make_wcopy(k_idx, n, buf):
            return pltpu.make_async_copy(
                w_hbm.at[pl.ds(k_idx * k_blk, k_blk),
                         pl.ds(n * n_tile, n_tile)],
                wbuf_ref.at[buf], w_sems.at[buf])

        def sweep(slot, first, last, amax_in):
            k_idx = lax.rem(me + slot, N_DEV)
            xsrc = send_ref.at[0] if slot == 0 else comm_ref.at[slot - 1]
            make_wcopy(k_idx, 0, 0).start()

            def step(n, amax):
                buf = lax.rem(n, 2)
                make_wcopy(k_idx, n, buf).wait()

                @pl.when(n + 1 < n_tiles)
                def _():
                    make_wcopy(k_idx, n + 1, 1 - buf).start()

                acc = lax.dot_general(
                    xsrc[...], wbuf_ref[buf].astype(jnp.bfloat16),
                    (((1,), (0,)), ((), ())),
                    preferred_element_type=jnp.float32)
                nds = pl.ds(n * n_tile, n_tile)
                if not first:
                    acc = acc + y_ref[:, nds].astype(jnp.float32)
                if last:
                    acc = jnp.maximum(acc, 0.0)
                    amax = jnp.maximum(amax, jnp.max(acc))
                y_ref[:, nds] = acc.astype(jnp.bfloat16)
                return amax

            return lax.fori_loop(0, n_tiles, step, amax_in)

        with jax.named_scope("sweep0"):
            sweep(0, True, False, jnp.float32(0.0))

        with jax.named_scope("diag_send"):
            rdma2 = make_rdma(2)
            rdma2.start()

        with jax.named_scope("wait_slot3"):
            rdma1.wait_recv()
        with jax.named_scope("sweep3"):
            sweep(3, False, False, jnp.float32(0.0))
        with jax.named_scope("wait_slot1"):
            rdma3.wait_recv()
        with jax.named_scope("sweep1"):
            sweep(1, False, False, jnp.float32(0.0))
        with jax.named_scope("wait_slot2"):
            rdma2.wait_recv()
        with jax.named_scope("sweep2"):
            amax = sweep(2, False, True, jnp.float32(0.0))

        with jax.named_scope("amax_xchg"):
            rdma1.wait_send()
            rdma3.wait_send()
            rdma2.wait_send()

            amax_src_ref[...] = jnp.full((8, 128), amax, jnp.float32)
            amax_comm_ref[0] = amax_src_ref[...]
            a_rdmas = []
            for t in range(1, N_DEV):
                target = lax.rem(me + t, N_DEV)
                slot = N_DEV - t
                r = pltpu.make_async_remote_copy(
                    src_ref=amax_src_ref,
                    dst_ref=amax_comm_ref.at[slot],
                    send_sem=amax_send_sems.at[slot],
                    recv_sem=amax_recv_sems.at[slot],
                    device_id=(target,),
                    device_id_type=pl.DeviceIdType.MESH,
                )
                r.start()
                a_rdmas.append(r)
            for r in a_rdmas:
                r.wait()

        g_amax = jnp.max(amax_comm_ref[...])
        inv = 127.0 / g_amax
        scale = g_amax / 127.0

        with jax.named_scope("quant"):
            def q_step(n, carry):
                nds = pl.ds(n * n_tile, n_tile)
                yt = out_ref[:, nds].astype(jnp.float32)
                q = jnp.clip(jnp.round(yt * inv), -127.0, 127.0)
                out_ref[:, nds] = (q * scale).astype(jnp.bfloat16)
                return carry

            lax.fori_loop(0, n_tiles, q_step, jnp.int32(0))

    return pl.pallas_call(
        body,
        out_shape=jax.ShapeDtypeStruct((m_blk, n_tot), jnp.bfloat16),
        in_specs=[pl.BlockSpec(memory_space=pltpu.VMEM),
                  pl.BlockSpec(memory_space=pl.ANY)],
        out_specs=pl.BlockSpec(memory_space=pltpu.VMEM),
        scratch_shapes=[
            pltpu.VMEM((N_DEV, m_blk, k_blk), jnp.bfloat16),
            pltpu.VMEM((3, m_blk, k_blk), jnp.bfloat16),
            pltpu.VMEM((2, k_blk, n_tile), jnp.float32),
            pltpu.VMEM((8, 128), jnp.float32),
            pltpu.VMEM((N_DEV, 8, 128), jnp.float32),
            pltpu.SemaphoreType.DMA((N_DEV,)),
            pltpu.SemaphoreType.DMA((N_DEV,)),
            pltpu.SemaphoreType.DMA((N_DEV,)),
            pltpu.SemaphoreType.DMA((N_DEV,)),
            pltpu.SemaphoreType.DMA((2,)),
        ],
        compiler_params=pltpu.CompilerParams(
            collective_id=0, vmem_limit_bytes=64 * 1024 * 1024),
    )(x, w_mat)


# device time: 127437 ns/iter; 1.0430x vs baseline; 1.0343x over previous
import jax
import jax.numpy as jnp
from jax import lax
from jax.experimental import pallas as pl
from jax.experimental.pallas import tpu as pltpu

N_DEV = 4


def kernel(x, w_mat):
    k_tot, k_blk = x.shape
    _, n_tot = w_mat.shape
    m_blk = k_tot // N_DEV
    n_tile = 1024
    n_tiles = n_tot // n_tile

    def body(x_hbm, w_hbm, out_hbm,
             xstage_ref, send_ref, comm_ref, y_ref, wbuf_ref, wpair_ref,
             ostage_ref, amax_src_ref, amax_comm_ref,
             send_sems, recv_sems, amax_send_sems, amax_recv_sems,
             xl_sems, w_sems, wp_sems, out_sems):
        me = lax.axis_index("i")

        scope_entry = jax.named_scope("entry")
        scope_entry.__enter__()
        barrier = pltpu.get_barrier_semaphore()
        for t in range(1, N_DEV):
            peer = lax.rem(me + t, N_DEV)
            pl.semaphore_signal(barrier, inc=1, device_id=(peer,),
                                device_id_type=pl.DeviceIdType.MESH)
        pl.semaphore_wait(barrier, N_DEV - 1)

        def xload(t, buf):
            rows = lax.rem(me + t, N_DEV) * m_blk
            return pltpu.make_async_copy(
                x_hbm.at[pl.ds(rows, m_blk), :], xstage_ref.at[buf],
                xl_sems.at[buf])

        def make_rdma(t):
            target = lax.rem(me + t, N_DEV)
            slot = N_DEV - t
            return pltpu.make_async_remote_copy(
                src_ref=send_ref.at[t],
                dst_ref=comm_ref.at[slot - 1],
                send_sem=send_sems.at[t],
                recv_sem=recv_sems.at[slot],
                device_id=(target,),
                device_id_type=pl.DeviceIdType.MESH,
            )

        cp1 = xload(1, 0)
        cp1.start()
        cp3 = xload(3, 1)
        cp3.start()

        cp1.wait()
        send_ref[1] = xstage_ref[0].astype(jnp.bfloat16)
        cp0 = xload(0, 0)
        cp0.start()
        rdma1 = make_rdma(1)
        rdma1.start()

        cp3.wait()
        send_ref[3] = xstage_ref[1].astype(jnp.bfloat16)
        cp2 = xload(2, 1)
        cp2.start()
        rdma3 = make_rdma(3)
        rdma3.start()

        cp0.wait()
        send_ref[0] = xstage_ref[0].astype(jnp.bfloat16)
        scope_entry.__exit__(None, None, None)

        def make_wcopy(k_idx, n, buf):
            return pltpu.make_async_copy(
                w_hbm.at[pl.ds(k_idx * k_blk, k_blk),
                         pl.ds(n * n_tile, n_tile)],
                wbuf_ref.at[buf], w_sems.at[buf])

        def sweep(slot, first, last, amax_in):
            k_idx = lax.rem(me + slot, N_DEV)
            xsrc = send_ref.at[0] if slot == 0 else comm_ref.at[slot - 1]
            make_wcopy(k_idx, 0, 0).start()

            def step(n, amax):
                buf = lax.rem(n, 2)
                make_wcopy(k_idx, n, buf).wait()

                @pl.when(n + 1 < n_tiles)
                def _():
                    make_wcopy(k_idx, n + 1, 1 - buf).start()

                acc = lax.dot_general(
                    xsrc[...], wbuf_ref[buf].astype(jnp.bfloat16),
                    (((1,), (0,)), ((), ())),
                    preferred_element_type=jnp.float32)
                nds = pl.ds(n * n_tile, n_tile)
                if not first:
                    acc = acc + y_ref[:, nds].astype(jnp.float32)
                if last:
                    acc = jnp.maximum(acc, 0.0)
                    amax = jnp.maximum(amax, jnp.max(acc))
                y_ref[:, nds] = acc.astype(jnp.bfloat16)
                return amax

            return lax.fori_loop(0, n_tiles, step, amax_in)

        p_tile = 512
        p_tiles = n_tot // p_tile

        def make_wpcopy(k_idx, n, buf):
            return pltpu.make_async_copy(
                w_hbm.at[pl.ds(k_idx * k_blk, k_blk),
                         pl.ds(n * p_tile, p_tile)],
                wpair_ref.at[buf], wp_sems.at[buf])

        def sweep_pair():
            kA = lax.rem(me + 3, N_DEV)
            kB = lax.rem(me + 1, N_DEV)
            make_wpcopy(kA, 0, 0).start()
            make_wpcopy(kB, 0, 1).start()

            def step(n, carry):
                pair = lax.rem(n, 2)
                bA = 2 * pair
                bB = 2 * pair + 1
                make_wpcopy(kA, n, bA).wait()
                make_wpcopy(kB, n, bB).wait()

                @pl.when(n + 1 < p_tiles)
                def _():
                    make_wpcopy(kA, n + 1, 2 * (1 - pair)).start()
                    make_wpcopy(kB, n + 1, 2 * (1 - pair) + 1).start()

                accA = lax.dot_general(
                    comm_ref[2], wpair_ref[bA].astype(jnp.bfloat16),
                    (((1,), (0,)), ((), ())),
                    preferred_element_type=jnp.float32)
                accB = lax.dot_general(
                    comm_ref[0], wpair_ref[bB].astype(jnp.bfloat16),
                    (((1,), (0,)), ((), ())),
                    preferred_element_type=jnp.float32)
                nds = pl.ds(n * p_tile, p_tile)
                acc = accA + accB + y_ref[:, nds].astype(jnp.float32)
                y_ref[:, nds] = acc.astype(jnp.bfloat16)
                return carry

            lax.fori_loop(0, p_tiles, step, jnp.int32(0))

        with jax.named_scope("sweep0"):
            sweep(0, True, False, jnp.float32(0.0))

        with jax.named_scope("diag_send"):
            cp2.wait()
            send_ref[2] = xstage_ref[1].astype(jnp.bfloat16)
            rdma2 = make_rdma(2)
            rdma2.start()

        with jax.named_scope("wait_slot31"):
            rdma1.wait_recv()
            rdma3.wait_recv()
        with jax.named_scope("sweep31"):
            sweep_pair()
        with jax.named_scope("wait_slot2"):
            rdma2.wait_recv()
        with jax.named_scope("sweep2"):
            amax = sweep(2, False, True, jnp.float32(0.0))

        with jax.named_scope("amax_xchg"):
            rdma1.wait_send()
            rdma3.wait_send()
            rdma2.wait_send()

            amax_src_ref[...] = jnp.full((8, 128), amax, jnp.float32)
            amax_comm_ref[0] = amax_src_ref[...]
            a_rdmas = []
            for t in range(1, N_DEV):
                target = lax.rem(me + t, N_DEV)
                slot = N_DEV - t
                r = pltpu.make_async_remote_copy(
                    src_ref=amax_src_ref,
                    dst_ref=amax_comm_ref.at[slot],
                    send_sem=amax_send_sems.at[slot],
                    recv_sem=amax_recv_sems.at[slot],
                    device_id=(target,),
                    device_id_type=pl.DeviceIdType.MESH,
                )
                r.start()
                a_rdmas.append(r)
            for r in a_rdmas:
                r.wait()

        g_amax = jnp.max(amax_comm_ref[...])
        inv = 127.0 / g_amax
        scale = g_amax / 127.0

        def make_ocopy(n, buf):
            return pltpu.make_async_copy(
                ostage_ref.at[buf], out_hbm.at[:, pl.ds(n * p_tile, p_tile)],
                out_sems.at[buf])

        with jax.named_scope("quant"):
            for n in range(p_tiles):
                buf = n % 2
                if n >= 2:
                    make_ocopy(n - 2, buf).wait()
                yt = y_ref[:, pl.ds(n * p_tile, p_tile)].astype(jnp.float32)
                q = jnp.clip(jnp.round(yt * inv), -127.0, 127.0)
                ostage_ref[buf] = (q * scale).astype(jnp.bfloat16)
                make_ocopy(n, buf).start()

            make_ocopy(p_tiles - 2, (p_tiles - 2) % 2).wait()
            make_ocopy(p_tiles - 1, (p_tiles - 1) % 2).wait()

    return pl.pallas_call(
        body,
        out_shape=jax.ShapeDtypeStruct((m_blk, n_tot), jnp.bfloat16),
        in_specs=[pl.BlockSpec(memory_space=pl.ANY),
                  pl.BlockSpec(memory_space=pl.ANY)],
        out_specs=pl.BlockSpec(memory_space=pl.ANY),
        scratch_shapes=[
            pltpu.VMEM((2, m_blk, k_blk), jnp.float32),
            pltpu.VMEM((N_DEV, m_blk, k_blk), jnp.bfloat16),
            pltpu.VMEM((3, m_blk, k_blk), jnp.bfloat16),
            pltpu.VMEM((m_blk, n_tot), jnp.bfloat16),
            pltpu.VMEM((2, k_blk, n_tile), jnp.float32),
            pltpu.VMEM((4, k_blk, 512), jnp.float32),
            pltpu.VMEM((2, m_blk, 512), jnp.bfloat16),
            pltpu.VMEM((8, 128), jnp.float32),
            pltpu.VMEM((N_DEV, 8, 128), jnp.float32),
            pltpu.SemaphoreType.DMA((N_DEV,)),
            pltpu.SemaphoreType.DMA((N_DEV,)),
            pltpu.SemaphoreType.DMA((N_DEV,)),
            pltpu.SemaphoreType.DMA((N_DEV,)),
            pltpu.SemaphoreType.DMA((2,)),
            pltpu.SemaphoreType.DMA((2,)),
            pltpu.SemaphoreType.DMA((4,)),
            pltpu.SemaphoreType.DMA((2,)),
        ],
        compiler_params=pltpu.CompilerParams(
            collective_id=0, vmem_limit_bytes=64 * 1024 * 1024),
    )(x, w_mat)


# device time: 126489 ns/iter; 1.0508x vs baseline; 1.0075x over previous
import jax
import jax.numpy as jnp
from jax import lax
from jax.experimental import pallas as pl
from jax.experimental.pallas import tpu as pltpu

N_DEV = 4


def kernel(x, w_mat):
    k_tot, k_blk = x.shape
    _, n_tot = w_mat.shape
    m_blk = k_tot // N_DEV
    n_tile = 1024
    n_tiles = n_tot // n_tile

    def body(x_hbm, w_hbm, out_hbm,
             xstage_ref, send_ref, comm_ref, y_ref, wbuf_ref,
             ostage_ref, amax_src_ref, amax_comm_ref,
             send_sems, recv_sems, amax_send_sems, amax_recv_sems,
             xl_sems, w_sems, out_sems):
        me = lax.axis_index("i")

        scope_entry = jax.named_scope("entry")
        scope_entry.__enter__()
        barrier = pltpu.get_barrier_semaphore()
        for t in range(1, N_DEV):
            peer = lax.rem(me + t, N_DEV)
            pl.semaphore_signal(barrier, inc=1, device_id=(peer,),
                                device_id_type=pl.DeviceIdType.MESH)
        pl.semaphore_wait(barrier, N_DEV - 1)

        def xload(t, buf):
            rows = lax.rem(me + t, N_DEV) * m_blk
            return pltpu.make_async_copy(
                x_hbm.at[pl.ds(rows, m_blk), :], xstage_ref.at[buf],
                xl_sems.at[buf])

        def make_rdma(t):
            target = lax.rem(me + t, N_DEV)
            slot = N_DEV - t
            return pltpu.make_async_remote_copy(
                src_ref=send_ref.at[t],
                dst_ref=comm_ref.at[slot - 1],
                send_sem=send_sems.at[t],
                recv_sem=recv_sems.at[slot],
                device_id=(target,),
                device_id_type=pl.DeviceIdType.MESH,
            )

        cp1 = xload(1, 0)
        cp1.start()
        cp3 = xload(3, 1)
        cp3.start()

        cp1.wait()
        send_ref[1] = xstage_ref[0].astype(jnp.bfloat16)
        cp0 = xload(0, 0)
        cp0.start()
        rdma1 = make_rdma(1)
        rdma1.start()

        cp3.wait()
        send_ref[3] = xstage_ref[1].astype(jnp.bfloat16)
        cp2 = xload(2, 1)
        cp2.start()
        rdma3 = make_rdma(3)
        rdma3.start()

        cp0.wait()
        send_ref[0] = xstage_ref[0].astype(jnp.bfloat16)
        scope_entry.__exit__(None, None, None)

        def make_wcopy(k_idx, n, buf):
            return pltpu.make_async_copy(
                w_hbm.at[pl.ds(k_idx * k_blk, k_blk),
                         pl.ds(n * n_tile, n_tile)],
                wbuf_ref.at[buf], w_sems.at[buf])

        def sweep(slot, first, last, amax_in):
            k_idx = lax.rem(me + slot, N_DEV)
            xsrc = send_ref.at[0] if slot == 0 else comm_ref.at[slot - 1]
            make_wcopy(k_idx, 0, 0).start()

            def step(n, amax):
                buf = lax.rem(n, 2)
                make_wcopy(k_idx, n, buf).wait()

                @pl.when(n + 1 < n_tiles)
                def _():
                    make_wcopy(k_idx, n + 1, 1 - buf).start()

                acc = lax.dot_general(
                    xsrc[...], wbuf_ref[buf].astype(jnp.bfloat16),
                    (((1,), (0,)), ((), ())),
                    preferred_element_type=jnp.float32)
                nds = pl.ds(n * n_tile, n_tile)
                if not first:
                    acc = acc + y_ref[:, nds].astype(jnp.float32)
                if last:
                    acc = jnp.maximum(acc, 0.0)
                    amax = jnp.maximum(amax, jnp.max(acc))
                y_ref[:, nds] = acc.astype(jnp.bfloat16)
                return amax

            return lax.fori_loop(0, n_tiles, step, amax_in)

        def sweep_pair():
            kA = lax.rem(me + 3, N_DEV)
            kB = lax.rem(me + 1, N_DEV)
            make_wcopy(kA, 0, 0).start()
            make_wcopy(kB, 0, 1).start()

            def step(n, carry):
                pair = lax.rem(n, 2)
                bA = 2 * pair
                bB = 2 * pair + 1
                make_wcopy(kA, n, bA).wait()
                make_wcopy(kB, n, bB).wait()

                @pl.when(n + 1 < n_tiles)
                def _():
                    make_wcopy(kA, n + 1, 2 * (1 - pair)).start()
                    make_wcopy(kB, n + 1, 2 * (1 - pair) + 1).start()

                accA = lax.dot_general(
                    comm_ref[2], wbuf_ref[bA].astype(jnp.bfloat16),
                    (((1,), (0,)), ((), ())),
                    preferred_element_type=jnp.float32)
                accB = lax.dot_general(
                    comm_ref[0], wbuf_ref[bB].astype(jnp.bfloat16),
                    (((1,), (0,)), ((), ())),
                    preferred_element_type=jnp.float32)
                nds = pl.ds(n * n_tile, n_tile)
                acc = accA + accB + y_ref[:, nds].astype(jnp.float32)
                y_ref[:, nds] = acc.astype(jnp.bfloat16)
                return carry

            lax.fori_loop(0, n_tiles, step, jnp.int32(0))

        with jax.named_scope("sweep0"):
            sweep(0, True, False, jnp.float32(0.0))

        with jax.named_scope("diag_send"):
            cp2.wait()
            send_ref[2] = xstage_ref[1].astype(jnp.bfloat16)
            rdma2 = make_rdma(2)
            rdma2.start()

        with jax.named_scope("wait_slot31"):
            rdma1.wait_recv()
            rdma3.wait_recv()
        with jax.named_scope("sweep31"):
            sweep_pair()
        with jax.named_scope("wait_slot2"):
            rdma2.wait_recv()
        with jax.named_scope("sweep2"):
            amax = sweep(2, False, True, jnp.float32(0.0))

        with jax.named_scope("amax_xchg"):
            rdma1.wait_send()
            rdma3.wait_send()
            rdma2.wait_send()

            amax_src_ref[...] = jnp.full((8, 128), amax, jnp.float32)
            amax_comm_ref[0] = amax_src_ref[...]
            a_rdmas = []
            for t in range(1, N_DEV):
                target = lax.rem(me + t, N_DEV)
                slot = N_DEV - t
                r = pltpu.make_async_remote_copy(
                    src_ref=amax_src_ref,
                    dst_ref=amax_comm_ref.at[slot],
                    send_sem=amax_send_sems.at[slot],
                    recv_sem=amax_recv_sems.at[slot],
                    device_id=(target,),
                    device_id_type=pl.DeviceIdType.MESH,
                )
                r.start()
                a_rdmas.append(r)
            for r in a_rdmas:
                r.wait()

        g_amax = jnp.max(amax_comm_ref[...])
        inv = 127.0 / g_amax
        scale = g_amax / 127.0

        q_tile = 512
        q_tiles = n_tot // q_tile

        def make_ocopy(n, buf):
            return pltpu.make_async_copy(
                ostage_ref.at[buf], out_hbm.at[:, pl.ds(n * q_tile, q_tile)],
                out_sems.at[buf])

        with jax.named_scope("quant"):
            for n in range(q_tiles):
                buf = n % 2
                if n >= 2:
                    make_ocopy(n - 2, buf).wait()
                yt = y_ref[:, pl.ds(n * q_tile, q_tile)].astype(jnp.float32)
                q = jnp.clip(jnp.round(yt * inv), -127.0, 127.0)
                ostage_ref[buf] = (q * scale).astype(jnp.bfloat16)
                make_ocopy(n, buf).start()

            make_ocopy(q_tiles - 2, (q_tiles - 2) % 2).wait()
            make_ocopy(q_tiles - 1, (q_tiles - 1) % 2).wait()

    return pl.pallas_call(
        body,
        out_shape=jax.ShapeDtypeStruct((m_blk, n_tot), jnp.bfloat16),
        in_specs=[pl.BlockSpec(memory_space=pl.ANY),
                  pl.BlockSpec(memory_space=pl.ANY)],
        out_specs=pl.BlockSpec(memory_space=pl.ANY),
        scratch_shapes=[
            pltpu.VMEM((2, m_blk, k_blk), jnp.float32),
            pltpu.VMEM((N_DEV, m_blk, k_blk), jnp.bfloat16),
            pltpu.VMEM((3, m_blk, k_blk), jnp.bfloat16),
            pltpu.VMEM((m_blk, n_tot), jnp.bfloat16),
            pltpu.VMEM((4, k_blk, n_tile), jnp.float32),
            pltpu.VMEM((2, m_blk, 512), jnp.bfloat16),
            pltpu.VMEM((8, 128), jnp.float32),
            pltpu.VMEM((N_DEV, 8, 128), jnp.float32),
            pltpu.SemaphoreType.DMA((N_DEV,)),
            pltpu.SemaphoreType.DMA((N_DEV,)),
            pltpu.SemaphoreType.DMA((N_DEV,)),
            pltpu.SemaphoreType.DMA((N_DEV,)),
            pltpu.SemaphoreType.DMA((2,)),
            pltpu.SemaphoreType.DMA((4,)),
            pltpu.SemaphoreType.DMA((2,)),
        ],
        compiler_params=pltpu.CompilerParams(
            collective_id=0, vmem_limit_bytes=64 * 1024 * 1024),
    )(x, w_mat)
